# baseline (speedup 1.0000x reference)
import jax
import jax.numpy as jnp
from jax import lax
from jax.experimental import pallas as pl
from jax.experimental.pallas import tpu as pltpu
from jax.experimental.pallas import tpu_sc as plsc

NC = 2
NS = 16
L = 16
NW = NC * NS
BATCH = 16384
K = 64
N_LIVE = 1000
BPW = BATCH // NW
NBLK = BPW // L
CHUNK = 128
NCHUNK = BPW // CHUNK


def _tec_body(hs_hbm, ls_hbm, ts_hbm, emb_e_hbm, emb_r_hbm, out_hbm,
              hs_v, ls_v, ts_v, eh_v, el_v, et_v, out_v, sem):
    cid = lax.axis_index("c")
    sid = lax.axis_index("s")
    wid = sid * NC + cid
    base = wid * BPW

    pltpu.sync_copy(hs_hbm.at[pl.ds(base, BPW)], hs_v)
    pltpu.sync_copy(ls_hbm.at[pl.ds(base, BPW)], ls_v)
    pltpu.sync_copy(ts_hbm.at[pl.ds(base, BPW)], ts_v)

    cps = []
    for j in range(NCHUNK):
        s = pl.ds(j * CHUNK, CHUNK)
        cps.append(pltpu.async_copy(
            emb_e_hbm.at[hs_v.at[s]], eh_v.at[s], sem))
        cps.append(pltpu.async_copy(
            emb_r_hbm.at[ls_v.at[s]], el_v.at[s], sem))
        cps.append(pltpu.async_copy(
            emb_e_hbm.at[ts_v.at[s]], et_v.at[s], sem))
    for cp in cps:
        cp.wait()

    def block(b, carry):
        rows = b * L + lax.iota(jnp.int32, L)
        acc = jnp.zeros((L,), jnp.float32)
        for c in range(K):
            col = jnp.full((L,), c, jnp.int32)
            eh = plsc.load_gather(eh_v, [rows, col])
            el = plsc.load_gather(el_v, [rows, col])
            et = plsc.load_gather(et_v, [rows, col])
            d = eh + el - et
            acc = acc + d * d
        i = plsc.bitcast(acc, jnp.int32)
        i = jnp.int32(0x5F3759DF) - lax.shift_right_logical(i, 1)
        y = plsc.bitcast(i, jnp.float32)
        half = acc * jnp.float32(0.5)
        for _ in range(3):
            y = y * (jnp.float32(1.5) - half * y * y)
        out_v[pl.ds(b * L, L)] = acc * y
        return carry

    lax.fori_loop(0, NBLK, block, 0)
    pltpu.sync_copy(out_v, out_hbm.at[pl.ds(base, BPW)])


_sc_call = pl.kernel(
    _tec_body,
    out_type=jax.ShapeDtypeStruct((BATCH,), jnp.float32),
    mesh=plsc.VectorSubcoreMesh(
        core_axis_name="c", subcore_axis_name="s",
        num_cores=NC, num_subcores=NS),
    scratch_types=[
        pltpu.VMEM((BPW,), jnp.int32),
        pltpu.VMEM((BPW,), jnp.int32),
        pltpu.VMEM((BPW,), jnp.int32),
        pltpu.VMEM((BPW, K), jnp.float32),
        pltpu.VMEM((BPW, K), jnp.float32),
        pltpu.VMEM((BPW, K), jnp.float32),
        pltpu.VMEM((BPW,), jnp.float32),
        pltpu.SemaphoreType.DMA,
    ],
    compiler_params=pltpu.CompilerParams(
        needs_layout_passes=False, use_tc_tiling_on_sc=False),
)


@jax.jit
def kernel(X, emb_E, emb_R):
    hs = X[:, 0].astype(jnp.int32)
    ls = X[:, 1].astype(jnp.int32)
    ts = X[:, 2].astype(jnp.int32)
    f = _sc_call(hs, ls, ts, emb_E[:N_LIVE], emb_R)
    return f.reshape(-1, 1)


# speedup vs baseline: 1.8210x; 1.8210x over previous
import jax
import jax.numpy as jnp
from jax import lax
from jax.experimental import pallas as pl
from jax.experimental.pallas import tpu as pltpu
from jax.experimental.pallas import tpu_sc as plsc

NC = 2
NS = 16
L = 16
NW = NC * NS
BATCH = 16384
K = 64
N_LIVE = 1000
BPW = BATCH // NW
NBLK = BPW // L
CHUNK = 128
NCHUNK = BPW // CHUNK


def _tec_body(hs_hbm, ls_hbm, ts_hbm, emb_e_hbm, emb_r_hbm, out_hbm,
              hs_v, ls_v, ts_v, eh_v, el_v, et_v, out_v, sem):
    cid = lax.axis_index("c")
    sid = lax.axis_index("s")
    wid = sid * NC + cid
    base = wid * BPW

    pltpu.sync_copy(hs_hbm.at[pl.ds(base, BPW)], hs_v)
    pltpu.sync_copy(ls_hbm.at[pl.ds(base, BPW)], ls_v)
    pltpu.sync_copy(ts_hbm.at[pl.ds(base, BPW)], ts_v)

    cps = []
    for j in range(NCHUNK):
        s = pl.ds(j * CHUNK, CHUNK)
        cps.append(pltpu.async_copy(
            emb_e_hbm.at[hs_v.at[s]], eh_v.at[s], sem))
        cps.append(pltpu.async_copy(
            emb_r_hbm.at[ls_v.at[s]], el_v.at[s], sem))
        cps.append(pltpu.async_copy(
            emb_e_hbm.at[ts_v.at[s]], et_v.at[s], sem))
    for cp in cps:
        cp.wait()

    def block(b, carry):
        lane = lax.iota(jnp.int32, L)
        rows = b * L + lane
        acc = jnp.zeros((L,), jnp.float32)
        for c in range(K):
            col = (lane + c) & (K - 1)
            eh = plsc.load_gather(eh_v, [rows, col])
            el = plsc.load_gather(el_v, [rows, col])
            et = plsc.load_gather(et_v, [rows, col])
            d = eh + el - et
            acc = acc + d * d
        i = plsc.bitcast(acc, jnp.int32)
        i = jnp.int32(0x5F3759DF) - lax.shift_right_logical(i, 1)
        y = plsc.bitcast(i, jnp.float32)
        half = acc * jnp.float32(0.5)
        for _ in range(3):
            y = y * (jnp.float32(1.5) - half * y * y)
        out_v[pl.ds(b * L, L)] = acc * y
        return carry

    lax.fori_loop(0, NBLK, block, 0)
    pltpu.sync_copy(out_v, out_hbm.at[pl.ds(base, BPW)])


_sc_call = pl.kernel(
    _tec_body,
    out_type=jax.ShapeDtypeStruct((BATCH,), jnp.float32),
    mesh=plsc.VectorSubcoreMesh(
        core_axis_name="c", subcore_axis_name="s",
        num_cores=NC, num_subcores=NS),
    scratch_types=[
        pltpu.VMEM((BPW,), jnp.int32),
        pltpu.VMEM((BPW,), jnp.int32),
        pltpu.VMEM((BPW,), jnp.int32),
        pltpu.VMEM((BPW, K), jnp.float32),
        pltpu.VMEM((BPW, K), jnp.float32),
        pltpu.VMEM((BPW, K), jnp.float32),
        pltpu.VMEM((BPW,), jnp.float32),
        pltpu.SemaphoreType.DMA,
    ],
    compiler_params=pltpu.CompilerParams(
        needs_layout_passes=False, use_tc_tiling_on_sc=False),
)


@jax.jit
def kernel(X, emb_E, emb_R):
    hs = X[:, 0].astype(jnp.int32)
    ls = X[:, 1].astype(jnp.int32)
    ts = X[:, 2].astype(jnp.int32)
    f = _sc_call(hs, ls, ts, emb_E[:N_LIVE], emb_R)
    return f.reshape(-1, 1)
